# Initial kernel scaffold; baseline (speedup 1.0000x reference)
import jax
import jax.numpy as jnp
from jax.experimental import pallas as pl


def kernel(hidden_states, w_router, w13, w2):
    raise NotImplementedError("write your pallas kernel here")


# speedup vs baseline: 1.9067x; 1.9067x over previous
import functools

import jax
import jax.numpy as jnp
from jax.experimental import pallas as pl
from jax.experimental.pallas import tpu as pltpu


def _moe_dense_body(x_ref, wr_ref, w13_ref, w2_ref, out_ref, logits_ref,
                    combine_ref):
    e = pl.program_id(0)
    T, E = logits_ref.shape

    @pl.when(e == 0)
    def _router():
        x = x_ref[...]
        logits = jax.lax.dot_general(
            x, wr_ref[...], (((1,), (1,)), ((), ())),
            preferred_element_type=jnp.float32)
        logits_ref[...] = logits
        lane = jax.lax.broadcasted_iota(jnp.int32, (T, E), 1)
        m1 = jnp.max(logits, axis=1, keepdims=True)
        i1 = jnp.min(jnp.where(logits == m1, lane, E), axis=1, keepdims=True)
        masked = jnp.where(lane == i1, -jnp.inf, logits)
        m2 = jnp.max(masked, axis=1, keepdims=True)
        i2 = jnp.min(jnp.where(masked == m2, lane, E), axis=1, keepdims=True)
        w1 = 1.0 / (1.0 + jnp.exp(m2 - m1))
        w2c = 1.0 - w1
        combine_ref[...] = (jnp.where(lane == i1, w1, 0.0)
                            + jnp.where(lane == i2, w2c, 0.0))

    x = x_ref[...]
    w13 = w13_ref[0]
    h = jax.lax.dot_general(x, w13, (((1,), (1,)), ((), ())),
                            preferred_element_type=jnp.float32)
    dff = h.shape[1] // 2
    gate = h[:, :dff]
    up = h[:, dff:]
    act = gate * jax.lax.logistic(gate) * up
    y = jax.lax.dot_general(act, w2_ref[0], (((1,), (1,)), ((), ())),
                            preferred_element_type=jnp.float32)
    lane_e = jax.lax.broadcasted_iota(jnp.int32, (T, E), 1)
    coef = jnp.sum(jnp.where(lane_e == e, combine_ref[...], 0.0),
                   axis=1, keepdims=True)

    @pl.when(e == 0)
    def _init():
        out_ref[...] = coef * y

    @pl.when(e != 0)
    def _acc():
        out_ref[...] += coef * y


def kernel(hidden_states, w_router, w13, w2):
    T, D = hidden_states.shape
    E = w_router.shape[0]
    out, logits = pl.pallas_call(
        _moe_dense_body,
        grid=(E,),
        in_specs=[
            pl.BlockSpec((T, D), lambda e: (0, 0)),
            pl.BlockSpec((E, D), lambda e: (0, 0)),
            pl.BlockSpec((1, w13.shape[1], D), lambda e: (e, 0, 0)),
            pl.BlockSpec((1, D, w2.shape[2]), lambda e: (e, 0, 0)),
        ],
        out_specs=[
            pl.BlockSpec((T, D), lambda e: (0, 0)),
            pl.BlockSpec((T, E), lambda e: (0, 0)),
        ],
        out_shape=[
            jax.ShapeDtypeStruct((T, D), jnp.float32),
            jax.ShapeDtypeStruct((T, E), jnp.float32),
        ],
        scratch_shapes=[pltpu.VMEM((T, E), jnp.float32)],
    )(hidden_states, w_router, w13, w2)
    return out, logits
